# baseline (speedup 1.0000x reference)
import functools

import jax
import jax.numpy as jnp
from jax import lax
from jax.experimental import pallas as pl
from jax.experimental.pallas import tpu as pltpu
from jax.experimental.pallas import tpu_sc as plsc

N_NODES = 10000
N_EDGES = 320000
D_NODE = 128
D_EDGE = 64
DEGREE = 64

NC = 2
NS = 16
NW = NC * NS
G = 5
EG = N_EDGES // G
E_PER_WG = EG // NW
CHUNK = 80
NCG = E_PER_WG // CHUNK
ROWS_PER_TILE = 624
TAIL_OFF = 15 * ROWS_PER_TILE
TAIL_ROWS = N_NODES - TAIL_OFF


@functools.lru_cache(maxsize=None)
def _get_sc_mesh():
    return plsc.VectorSubcoreMesh(core_axis_name="c", subcore_axis_name="s")


@functools.lru_cache(maxsize=None)
def _get_sc_gather():
    @functools.partial(
        pl.kernel,
        out_type=jax.ShapeDtypeStruct((EG, 2 * D_NODE), jnp.float32),
        mesh=_get_sc_mesh(),
        scratch_types=[
            pltpu.VMEM((NCG, CHUNK), jnp.int32),
            pltpu.VMEM((NCG, CHUNK), jnp.int32),
            pltpu.VMEM((CHUNK, 2 * D_NODE), jnp.float32),
            pltpu.VMEM((CHUNK, 2 * D_NODE), jnp.float32),
            pltpu.SemaphoreType.DMA,
            pltpu.SemaphoreType.DMA,
            pltpu.SemaphoreType.DMA,
        ],
    )
    def _sc_gather(nf_hbm, srcm_hbm, dstm_hbm, out_hbm,
                   idxs, idxd, ra, rb, sem_a, sem_b, sem_w):
        wid = lax.axis_index("s") * NC + lax.axis_index("c")
        base = wid * E_PER_WG

        pltpu.sync_copy(srcm_hbm.at[wid], idxs)
        pltpu.sync_copy(dstm_hbm.at[wid], idxd)

        def rows(c):
            return pl.ds(base + c * CHUNK, CHUNK)

        def body(i, _):
            c0 = 2 * i
            c1 = 2 * i + 1
            g0s = pltpu.async_copy(nf_hbm.at[idxs.at[c0]], ra.at[:, pl.ds(0, D_NODE)], sem_a)
            g0d = pltpu.async_copy(nf_hbm.at[idxd.at[c0]], ra.at[:, pl.ds(D_NODE, D_NODE)], sem_a)
            g1s = pltpu.async_copy(nf_hbm.at[idxs.at[c1]], rb.at[:, pl.ds(0, D_NODE)], sem_b)
            g1d = pltpu.async_copy(nf_hbm.at[idxd.at[c1]], rb.at[:, pl.ds(D_NODE, D_NODE)], sem_b)
            g0s.wait()
            g0d.wait()
            w0 = pltpu.async_copy(ra, out_hbm.at[rows(c0)], sem_w)
            g1s.wait()
            g1d.wait()
            w1 = pltpu.async_copy(rb, out_hbm.at[rows(c1)], sem_w)
            w0.wait()
            w1.wait()

        lax.fori_loop(0, NCG // 2, body, None)

        ct = NCG - 1
        gts = pltpu.async_copy(nf_hbm.at[idxs.at[ct]], ra.at[:, pl.ds(0, D_NODE)], sem_a)
        gtd = pltpu.async_copy(nf_hbm.at[idxd.at[ct]], ra.at[:, pl.ds(D_NODE, D_NODE)], sem_a)
        gts.wait()
        gtd.wait()
        wt = pltpu.async_copy(ra, out_hbm.at[rows(ct)], sem_w)
        wt.wait()

    return _sc_gather


BE = 2560
NBG = EG // BE


def _mlp_body(vij_ref, ea_ref, ew_ref,
              Wv_lo_ref, Wv_hi_ref, W1ea_e_ref, b1e_ref, W2e_ref, b2e_ref,
              W1ea_n_ref, b1n_ref, W2n_ref, b2n_ref, WeWn_ref,
              ea_new_ref, feats_ref):
    f32 = jnp.float32
    bf = jnp.bfloat16
    v_lo = vij_ref[:, 0:D_NODE].astype(bf)
    v_hi = vij_ref[:, D_NODE:2 * D_NODE].astype(bf)
    ea = ea_ref[...]
    ea_bf = ea.astype(bf)
    ew = ew_ref[...].astype(bf)

    pre1 = (jnp.dot(v_lo, Wv_lo_ref[...], preferred_element_type=f32)
            + jnp.dot(v_hi, Wv_hi_ref[...], preferred_element_type=f32))
    ewp = jnp.dot(ew, WeWn_ref[...], preferred_element_type=f32)

    he = pre1[:, 0:128] + jnp.dot(ea_bf, W1ea_e_ref[...], preferred_element_type=f32)
    he = he + b1e_ref[...]
    he = he * jax.nn.sigmoid(he)
    s2e = jnp.dot(he.astype(bf), W2e_ref[...], preferred_element_type=f32) + b2e_ref[...]
    ue = s2e[:, 0:64]
    ue = ue * jax.nn.sigmoid(ue)
    ge = jax.nn.sigmoid(s2e[:, 64:128])
    ea_new = ea + ue * ge * ewp[:, 0:64]
    ea_new_ref[...] = ea_new

    hn = pre1[:, 128:256] + jnp.dot(ea_new.astype(bf), W1ea_n_ref[...],
                                    preferred_element_type=f32)
    hn = hn + b1n_ref[...]
    hn = hn * jax.nn.sigmoid(hn)
    s2n = jnp.dot(hn.astype(bf), W2n_ref[...], preferred_element_type=f32) + b2n_ref[...]
    un = s2n[:, 0:128]
    un = un * jax.nn.sigmoid(un)
    gn = jax.nn.sigmoid(s2n[:, 128:256])
    feats_ref[...] = un * gn * ewp[:, 64:192]


def _run_mlp(g, vij, ea, ew, Wv_lo, Wv_hi, W1ea_e, b1e, W2e, b2e, W1ea_n, b1n,
             W2n, b2n, WeWn):
    blk = lambda shape: pl.BlockSpec(shape, lambda i: (0,) * len(shape))
    gbs = lambda w: pl.BlockSpec((BE, w), lambda i: (i, 0))
    fbs = lambda w: pl.BlockSpec((BE, w), lambda i: (g * NBG + i, 0))
    return pl.pallas_call(
        _mlp_body,
        grid=(NBG,),
        in_specs=[
            gbs(256), fbs(64), fbs(64),
            blk((128, 256)), blk((128, 256)), blk((64, 128)), blk((1, 128)),
            blk((128, 128)), blk((1, 128)), blk((64, 128)), blk((1, 128)),
            blk((128, 256)), blk((1, 256)), blk((64, 192)),
        ],
        out_specs=[gbs(64), gbs(128)],
        out_shape=[
            jax.ShapeDtypeStruct((EG, D_EDGE), jnp.float32),
            jax.ShapeDtypeStruct((EG, D_NODE), jnp.float32),
        ],
    )(vij, ea, ew, Wv_lo, Wv_hi, W1ea_e, b1e, W2e, b2e, W1ea_n, b1n, W2n, b2n,
      WeWn)


@functools.lru_cache(maxsize=None)
def _get_sc_scatter():
    @functools.partial(
        pl.kernel,
        out_type=jax.ShapeDtypeStruct((NC, N_NODES, D_NODE), jnp.float32),
        mesh=_get_sc_mesh(),
        scratch_types=[
            pltpu.VMEM_SHARED((N_NODES, D_NODE), jnp.float32),
            pltpu.VMEM((NCG, CHUNK), jnp.int32),
            pltpu.VMEM((CHUNK, D_NODE), jnp.float32),
            pltpu.VMEM((CHUNK, D_NODE), jnp.float32),
            pltpu.SemaphoreType.DMA,
            pltpu.SemaphoreType.DMA,
        ],
    )
    def _sc_scatter(f0, f1, f2, f3, f4, srcm_hbm, nf_hbm, zeros_hbm, out_hbm,
                    acc, idxs, rowa, rowb, sem_a, sem_b):
        cid = lax.axis_index("c")
        sid = lax.axis_index("s")
        wid = sid * NC + cid
        base = wid * E_PER_WG
        roff = sid * ROWS_PER_TILE

        @pl.when(cid == 0)
        def _():
            pltpu.sync_copy(nf_hbm.at[pl.ds(roff, ROWS_PER_TILE)],
                            acc.at[pl.ds(roff, ROWS_PER_TILE)])

            @pl.when(sid == NS - 1)
            def _():
                pltpu.sync_copy(nf_hbm.at[pl.ds(TAIL_OFF + ROWS_PER_TILE, TAIL_ROWS - ROWS_PER_TILE)],
                                acc.at[pl.ds(TAIL_OFF + ROWS_PER_TILE, TAIL_ROWS - ROWS_PER_TILE)])

        @pl.when(cid != 0)
        def _():
            pltpu.sync_copy(zeros_hbm.at[pl.ds(roff, ROWS_PER_TILE)],
                            acc.at[pl.ds(roff, ROWS_PER_TILE)])

            @pl.when(sid == NS - 1)
            def _():
                pltpu.sync_copy(zeros_hbm.at[pl.ds(TAIL_OFF + ROWS_PER_TILE, TAIL_ROWS - ROWS_PER_TILE)],
                                acc.at[pl.ds(TAIL_OFF + ROWS_PER_TILE, TAIL_ROWS - ROWS_PER_TILE)])

        plsc.subcore_barrier()

        def rows(c):
            return pl.ds(base + c * CHUNK, CHUNK)

        for gi, feats_hbm in enumerate((f0, f1, f2, f3, f4)):
            pltpu.sync_copy(srcm_hbm.at[gi, wid], idxs)

            def body(i, _, feats_hbm=feats_hbm):
                c0 = 2 * i
                c1 = 2 * i + 1
                fa = pltpu.async_copy(feats_hbm.at[rows(c0)], rowa, sem_a)
                fb = pltpu.async_copy(feats_hbm.at[rows(c1)], rowb, sem_b)
                fa.wait()
                pltpu.sync_copy(rowa, acc.at[idxs.at[c0]], add=True)
                fb.wait()
                pltpu.sync_copy(rowb, acc.at[idxs.at[c1]], add=True)

            lax.fori_loop(0, NCG // 2, body, None)

            ct = NCG - 1
            ft = pltpu.async_copy(feats_hbm.at[rows(ct)], rowa, sem_a)
            ft.wait()
            pltpu.sync_copy(rowa, acc.at[idxs.at[ct]], add=True)

        plsc.subcore_barrier()
        pltpu.sync_copy(acc.at[pl.ds(roff, ROWS_PER_TILE)],
                        out_hbm.at[cid, pl.ds(roff, ROWS_PER_TILE)])

        @pl.when(sid == NS - 1)
        def _():
            pltpu.sync_copy(acc.at[pl.ds(TAIL_OFF + ROWS_PER_TILE, TAIL_ROWS - ROWS_PER_TILE)],
                            out_hbm.at[cid, pl.ds(TAIL_OFF + ROWS_PER_TILE, TAIL_ROWS - ROWS_PER_TILE)])

    return _sc_scatter


def _combine_body(p_ref, out_ref):
    out_ref[...] = p_ref[0] + p_ref[1]


def _run_combine(parts):
    nb = 10
    rb = N_NODES // nb
    return pl.pallas_call(
        _combine_body,
        grid=(nb,),
        in_specs=[pl.BlockSpec((NC, rb, D_NODE), lambda i: (0, i, 0))],
        out_specs=pl.BlockSpec((rb, D_NODE), lambda i: (i, 0)),
        out_shape=jax.ShapeDtypeStruct((N_NODES, D_NODE), jnp.float32),
    )(parts)


def kernel(node_features, edge_index, edge_attr, edge_weights,
           eW1, eb1, eW2, eb2, egW1, egb1, egW2, egb2,
           nW1, nb1, nW2, nb2, ngW1, ngb1, ngW2, ngb2,
           We, Wn):
    bf = jnp.bfloat16
    src = edge_index[0].astype(jnp.int32)
    dst = edge_index[1].astype(jnp.int32)
    srcm = src.reshape(G, NW, NCG, CHUNK)
    dstm = dst.reshape(G, NW, NCG, CHUNK)

    top = jnp.concatenate([eW1[0:128], egW1[0:128], nW1[0:128], ngW1[0:128]], axis=1)
    bot = jnp.concatenate([eW1[128:256], egW1[128:256], nW1[128:256], ngW1[128:256]], axis=1)
    Wv_lo = top.astype(bf)
    Wv_hi = bot.astype(bf)
    W1ea_e = jnp.concatenate([eW1[256:320], egW1[256:320]], axis=1).astype(bf)
    W1ea_n = jnp.concatenate([nW1[256:320], ngW1[256:320]], axis=1).astype(bf)
    b1e = jnp.concatenate([eb1, egb1])[None, :]
    b1n = jnp.concatenate([nb1, ngb1])[None, :]
    z64 = jnp.zeros((64, 64), jnp.float32)
    W2e = jnp.block([[eW2, z64], [z64, egW2]]).astype(bf)
    b2e = jnp.concatenate([eb2, egb2])[None, :]
    z64n = jnp.zeros((64, 128), jnp.float32)
    W2n = jnp.block([[nW2, z64n], [z64n, ngW2]]).astype(bf)
    b2n = jnp.concatenate([nb2, ngb2])[None, :]
    WeWn = jnp.concatenate([We, Wn], axis=1).astype(bf)

    gather = _get_sc_gather()
    ean_gs = []
    feats_gs = []
    for g in range(G):
        vij_g = gather(node_features, srcm[g], dstm[g])
        ean_g, feats_g = _run_mlp(g, vij_g, edge_attr, edge_weights,
                                  Wv_lo, Wv_hi, W1ea_e, b1e, W2e, b2e,
                                  W1ea_n, b1n, W2n, b2n, WeWn)
        ean_gs.append(ean_g)
        feats_gs.append(feats_g)

    ea_new = jnp.concatenate(ean_gs, axis=0)
    zeros = jnp.zeros((N_NODES, D_NODE), jnp.float32)
    parts = _get_sc_scatter()(*feats_gs, srcm, node_features, zeros)
    node_new = _run_combine(parts)
    return (node_new, ea_new)


# speedup vs baseline: 1.3983x; 1.3983x over previous
import functools

import jax
import jax.numpy as jnp
from jax import lax
from jax.experimental import pallas as pl
from jax.experimental.pallas import tpu as pltpu
from jax.experimental.pallas import tpu_sc as plsc

N_NODES = 10000
N_EDGES = 320000
D_NODE = 128
D_EDGE = 64
DEGREE = 64

NC = 2
NS = 16
NW = NC * NS
G = 5
EG = N_EDGES // G
E_PER_WG = EG // NW
CHUNK = 80
NCG = E_PER_WG // CHUNK
ROWS_PER_TILE = 624
TAIL_OFF = 15 * ROWS_PER_TILE
TAIL_ROWS = N_NODES - TAIL_OFF


@functools.lru_cache(maxsize=None)
def _get_sc_mesh():
    return plsc.VectorSubcoreMesh(core_axis_name="c", subcore_axis_name="s")


@functools.lru_cache(maxsize=None)
def _get_sc_gather():
    @functools.partial(
        pl.kernel,
        out_type=jax.ShapeDtypeStruct((EG, 2 * D_NODE), jnp.float32),
        mesh=_get_sc_mesh(),
        scratch_types=[
            pltpu.VMEM((NCG, CHUNK), jnp.int32),
            pltpu.VMEM((NCG, CHUNK), jnp.int32),
            pltpu.VMEM((CHUNK, 2 * D_NODE), jnp.float32),
            pltpu.VMEM((CHUNK, 2 * D_NODE), jnp.float32),
            pltpu.SemaphoreType.DMA,
            pltpu.SemaphoreType.DMA,
            pltpu.SemaphoreType.DMA,
        ],
    )
    def _sc_gather(nf_hbm, srcm_hbm, dstm_hbm, out_hbm,
                   idxs, idxd, ra, rb, sem_a, sem_b, sem_w):
        wid = lax.axis_index("s") * NC + lax.axis_index("c")
        base = wid * E_PER_WG

        pltpu.sync_copy(srcm_hbm.at[wid], idxs)
        pltpu.sync_copy(dstm_hbm.at[wid], idxd)

        def rows(c):
            return pl.ds(base + c * CHUNK, CHUNK)

        def body(i, _):
            c0 = 2 * i
            c1 = 2 * i + 1
            g0s = pltpu.async_copy(nf_hbm.at[idxs.at[c0]], ra.at[:, pl.ds(0, D_NODE)], sem_a)
            g0d = pltpu.async_copy(nf_hbm.at[idxd.at[c0]], ra.at[:, pl.ds(D_NODE, D_NODE)], sem_a)
            g1s = pltpu.async_copy(nf_hbm.at[idxs.at[c1]], rb.at[:, pl.ds(0, D_NODE)], sem_b)
            g1d = pltpu.async_copy(nf_hbm.at[idxd.at[c1]], rb.at[:, pl.ds(D_NODE, D_NODE)], sem_b)
            g0s.wait()
            g0d.wait()
            w0 = pltpu.async_copy(ra, out_hbm.at[rows(c0)], sem_w)
            g1s.wait()
            g1d.wait()
            w1 = pltpu.async_copy(rb, out_hbm.at[rows(c1)], sem_w)
            w0.wait()
            w1.wait()

        lax.fori_loop(0, NCG // 2, body, None)

        ct = NCG - 1
        gts = pltpu.async_copy(nf_hbm.at[idxs.at[ct]], ra.at[:, pl.ds(0, D_NODE)], sem_a)
        gtd = pltpu.async_copy(nf_hbm.at[idxd.at[ct]], ra.at[:, pl.ds(D_NODE, D_NODE)], sem_a)
        gts.wait()
        gtd.wait()
        wt = pltpu.async_copy(ra, out_hbm.at[rows(ct)], sem_w)
        wt.wait()

    return _sc_gather


BE = 2560
NBG = EG // BE


def _mlp_body(vij_ref, ea_t_ref, ew_t_ref,
              Wv_lo_ref, Wv_hi_ref, W1ea_e_ref, b1e_ref, W2e_ref, b2e_ref,
              W1ea_n_ref, b1n_ref, W2n_ref, b2n_ref, We_ref, Wn_ref,
              ea_new_t_ref, feats_ref):
    f32 = jnp.float32
    bf = jnp.bfloat16

    def dg(lhs, rhs, dims):
        return lax.dot_general(lhs, rhs, (dims, ((), ())),
                               preferred_element_type=f32)

    v_lo = vij_ref[:, 0:D_NODE].astype(bf)
    v_hi = vij_ref[:, D_NODE:2 * D_NODE].astype(bf)
    ea_t = ea_t_ref[...]
    ea_t_bf = ea_t.astype(bf)
    ew_t = ew_t_ref[...].astype(bf)

    pre1 = (jnp.dot(v_lo, Wv_lo_ref[...], preferred_element_type=f32)
            + jnp.dot(v_hi, Wv_hi_ref[...], preferred_element_type=f32))

    he = pre1[:, 0:128] + dg(ea_t_bf, W1ea_e_ref[...], ((0,), (0,)))
    he = he + b1e_ref[...]
    he = he * jax.nn.sigmoid(he)
    s2e_t = dg(W2e_ref[...], he.astype(bf), ((0,), (1,))) + b2e_ref[...]
    ue_t = s2e_t[0:64]
    ue_t = ue_t * jax.nn.sigmoid(ue_t)
    ge_t = jax.nn.sigmoid(s2e_t[64:128])
    ewe_t = dg(We_ref[...], ew_t, ((0,), (0,)))
    ea_new_t = ea_t + ue_t * ge_t * ewe_t
    ea_new_t_ref[...] = ea_new_t

    hn = pre1[:, 128:256] + dg(ea_new_t.astype(bf), W1ea_n_ref[...], ((0,), (0,)))
    hn = hn + b1n_ref[...]
    hn = hn * jax.nn.sigmoid(hn)
    s2n = jnp.dot(hn.astype(bf), W2n_ref[...], preferred_element_type=f32) + b2n_ref[...]
    un = s2n[:, 0:128]
    un = un * jax.nn.sigmoid(un)
    gn = jax.nn.sigmoid(s2n[:, 128:256])
    ewn = dg(ew_t, Wn_ref[...], ((0,), (0,)))
    feats_ref[...] = un * gn * ewn


def _run_mlp(g, vij, ea_t, ew_t, Wv_lo, Wv_hi, W1ea_e, b1e, W2e, b2e, W1ea_n,
             b1n, W2n, b2n, We, Wn):
    blk = lambda shape: pl.BlockSpec(shape, lambda i: (0,) * len(shape))
    gbs = lambda w: pl.BlockSpec((BE, w), lambda i: (i, 0))
    tgbs = pl.BlockSpec((D_EDGE, BE), lambda i: (0, i))
    tfbs = pl.BlockSpec((D_EDGE, BE), lambda i: (0, g * NBG + i))
    return pl.pallas_call(
        _mlp_body,
        grid=(NBG,),
        in_specs=[
            gbs(256), tfbs, tfbs,
            blk((128, 256)), blk((128, 256)), blk((64, 128)), blk((1, 128)),
            blk((128, 128)), blk((128, 1)), blk((64, 128)), blk((1, 128)),
            blk((128, 256)), blk((1, 256)), blk((64, 64)), blk((64, 128)),
        ],
        out_specs=[tgbs, gbs(128)],
        out_shape=[
            jax.ShapeDtypeStruct((D_EDGE, EG), jnp.float32),
            jax.ShapeDtypeStruct((EG, D_NODE), jnp.float32),
        ],
    )(vij, ea_t, ew_t, Wv_lo, Wv_hi, W1ea_e, b1e, W2e, b2e, W1ea_n, b1n, W2n,
      b2n, We, Wn)


@functools.lru_cache(maxsize=None)
def _get_sc_scatter():
    @functools.partial(
        pl.kernel,
        out_type=jax.ShapeDtypeStruct((NC, N_NODES, D_NODE), jnp.float32),
        mesh=_get_sc_mesh(),
        scratch_types=[
            pltpu.VMEM_SHARED((N_NODES, D_NODE), jnp.float32),
            pltpu.VMEM((NCG, CHUNK), jnp.int32),
            pltpu.VMEM((CHUNK, D_NODE), jnp.float32),
            pltpu.VMEM((CHUNK, D_NODE), jnp.float32),
            pltpu.SemaphoreType.DMA,
            pltpu.SemaphoreType.DMA,
        ],
    )
    def _sc_scatter(f0, f1, f2, f3, f4, srcm_hbm, nf_hbm, zeros_hbm, out_hbm,
                    acc, idxs, rowa, rowb, sem_a, sem_b):
        cid = lax.axis_index("c")
        sid = lax.axis_index("s")
        wid = sid * NC + cid
        base = wid * E_PER_WG
        roff = sid * ROWS_PER_TILE

        @pl.when(cid == 0)
        def _():
            pltpu.sync_copy(nf_hbm.at[pl.ds(roff, ROWS_PER_TILE)],
                            acc.at[pl.ds(roff, ROWS_PER_TILE)])

            @pl.when(sid == NS - 1)
            def _():
                pltpu.sync_copy(nf_hbm.at[pl.ds(TAIL_OFF + ROWS_PER_TILE, TAIL_ROWS - ROWS_PER_TILE)],
                                acc.at[pl.ds(TAIL_OFF + ROWS_PER_TILE, TAIL_ROWS - ROWS_PER_TILE)])

        @pl.when(cid != 0)
        def _():
            pltpu.sync_copy(zeros_hbm.at[pl.ds(roff, ROWS_PER_TILE)],
                            acc.at[pl.ds(roff, ROWS_PER_TILE)])

            @pl.when(sid == NS - 1)
            def _():
                pltpu.sync_copy(zeros_hbm.at[pl.ds(TAIL_OFF + ROWS_PER_TILE, TAIL_ROWS - ROWS_PER_TILE)],
                                acc.at[pl.ds(TAIL_OFF + ROWS_PER_TILE, TAIL_ROWS - ROWS_PER_TILE)])

        plsc.subcore_barrier()

        def rows(c):
            return pl.ds(base + c * CHUNK, CHUNK)

        for gi, feats_hbm in enumerate((f0, f1, f2, f3, f4)):
            pltpu.sync_copy(srcm_hbm.at[gi, wid], idxs)

            def body(i, _, feats_hbm=feats_hbm):
                c0 = 2 * i
                c1 = 2 * i + 1
                fa = pltpu.async_copy(feats_hbm.at[rows(c0)], rowa, sem_a)
                fb = pltpu.async_copy(feats_hbm.at[rows(c1)], rowb, sem_b)
                fa.wait()
                pltpu.sync_copy(rowa, acc.at[idxs.at[c0]], add=True)
                fb.wait()
                pltpu.sync_copy(rowb, acc.at[idxs.at[c1]], add=True)

            lax.fori_loop(0, NCG // 2, body, None)

            ct = NCG - 1
            ft = pltpu.async_copy(feats_hbm.at[rows(ct)], rowa, sem_a)
            ft.wait()
            pltpu.sync_copy(rowa, acc.at[idxs.at[ct]], add=True)

        plsc.subcore_barrier()
        pltpu.sync_copy(acc.at[pl.ds(roff, ROWS_PER_TILE)],
                        out_hbm.at[cid, pl.ds(roff, ROWS_PER_TILE)])

        @pl.when(sid == NS - 1)
        def _():
            pltpu.sync_copy(acc.at[pl.ds(TAIL_OFF + ROWS_PER_TILE, TAIL_ROWS - ROWS_PER_TILE)],
                            out_hbm.at[cid, pl.ds(TAIL_OFF + ROWS_PER_TILE, TAIL_ROWS - ROWS_PER_TILE)])

    return _sc_scatter


def _combine_body(p_ref, out_ref):
    out_ref[...] = p_ref[0] + p_ref[1]


def _run_combine(parts):
    nb = 10
    rb = N_NODES // nb
    return pl.pallas_call(
        _combine_body,
        grid=(nb,),
        in_specs=[pl.BlockSpec((NC, rb, D_NODE), lambda i: (0, i, 0))],
        out_specs=pl.BlockSpec((rb, D_NODE), lambda i: (i, 0)),
        out_shape=jax.ShapeDtypeStruct((N_NODES, D_NODE), jnp.float32),
    )(parts)


def kernel(node_features, edge_index, edge_attr, edge_weights,
           eW1, eb1, eW2, eb2, egW1, egb1, egW2, egb2,
           nW1, nb1, nW2, nb2, ngW1, ngb1, ngW2, ngb2,
           We, Wn):
    bf = jnp.bfloat16
    src = edge_index[0].astype(jnp.int32)
    dst = edge_index[1].astype(jnp.int32)
    srcm = src.reshape(G, NW, NCG, CHUNK)
    dstm = dst.reshape(G, NW, NCG, CHUNK)

    top = jnp.concatenate([eW1[0:128], egW1[0:128], nW1[0:128], ngW1[0:128]], axis=1)
    bot = jnp.concatenate([eW1[128:256], egW1[128:256], nW1[128:256], ngW1[128:256]], axis=1)
    Wv_lo = top.astype(bf)
    Wv_hi = bot.astype(bf)
    W1ea_e = jnp.concatenate([eW1[256:320], egW1[256:320]], axis=1).astype(bf)
    W1ea_n = jnp.concatenate([nW1[256:320], ngW1[256:320]], axis=1).astype(bf)
    b1e = jnp.concatenate([eb1, egb1])[None, :]
    b1n = jnp.concatenate([nb1, ngb1])[None, :]
    z64 = jnp.zeros((64, 64), jnp.float32)
    W2e = jnp.block([[eW2, z64], [z64, egW2]]).astype(bf)
    b2e = jnp.concatenate([eb2, egb2])[:, None]
    z64n = jnp.zeros((64, 128), jnp.float32)
    W2n = jnp.block([[nW2, z64n], [z64n, ngW2]]).astype(bf)
    b2n = jnp.concatenate([nb2, ngb2])[None, :]
    We_bf = We.astype(bf)
    Wn_bf = Wn.astype(bf)

    ea_t = edge_attr.T
    ew_t = edge_weights.T

    gather = _get_sc_gather()
    ean_t_gs = []
    feats_gs = []
    for g in range(G):
        vij_g = gather(node_features, srcm[g], dstm[g])
        ean_t_g, feats_g = _run_mlp(g, vij_g, ea_t, ew_t,
                                    Wv_lo, Wv_hi, W1ea_e, b1e, W2e, b2e,
                                    W1ea_n, b1n, W2n, b2n, We_bf, Wn_bf)
        ean_t_gs.append(ean_t_g)
        feats_gs.append(feats_g)

    ea_new = jnp.concatenate(ean_t_gs, axis=1).T
    zeros = jnp.zeros((N_NODES, D_NODE), jnp.float32)
    parts = _get_sc_scatter()(*feats_gs, srcm, node_features, zeros)
    node_new = _run_combine(parts)
    return (node_new, ea_new)


# speedup vs baseline: 1.5476x; 1.1068x over previous
import functools

import jax
import jax.numpy as jnp
from jax import lax
from jax.experimental import pallas as pl
from jax.experimental.pallas import tpu as pltpu
from jax.experimental.pallas import tpu_sc as plsc

N_NODES = 10000
N_EDGES = 320000
D_NODE = 128
D_EDGE = 64
DEGREE = 64

NC = 2
NS = 16
NW = NC * NS
G = 5
EG = N_EDGES // G
E_PER_WG = EG // NW
CHUNK = 80
NCG = E_PER_WG // CHUNK
ROWS_PER_TILE = 624
TAIL_OFF = 15 * ROWS_PER_TILE
TAIL_ROWS = N_NODES - TAIL_OFF


@functools.lru_cache(maxsize=None)
def _get_sc_mesh():
    return plsc.VectorSubcoreMesh(core_axis_name="c", subcore_axis_name="s")


@functools.lru_cache(maxsize=None)
def _get_sc_gather():
    @functools.partial(
        pl.kernel,
        out_type=jax.ShapeDtypeStruct((EG, 2 * D_NODE), jnp.float32),
        mesh=_get_sc_mesh(),
        scratch_types=[
            pltpu.VMEM((NCG, CHUNK), jnp.int32),
            pltpu.VMEM((NCG, CHUNK), jnp.int32),
            pltpu.VMEM((CHUNK, 2 * D_NODE), jnp.float32),
            pltpu.VMEM((CHUNK, 2 * D_NODE), jnp.float32),
            pltpu.SemaphoreType.DMA,
            pltpu.SemaphoreType.DMA,
            pltpu.SemaphoreType.DMA,
        ],
    )
    def _sc_gather(nf_hbm, srcm_hbm, dstm_hbm, out_hbm,
                   idxs, idxd, ra, rb, sem_a, sem_b, sem_w):
        wid = lax.axis_index("s") * NC + lax.axis_index("c")
        base = wid * E_PER_WG

        pltpu.sync_copy(srcm_hbm.at[wid], idxs)
        pltpu.sync_copy(dstm_hbm.at[wid], idxd)

        def rows(c):
            return pl.ds(base + c * CHUNK, CHUNK)

        def body(i, _):
            c0 = 2 * i
            c1 = 2 * i + 1
            g0s = pltpu.async_copy(nf_hbm.at[idxs.at[c0]], ra.at[:, pl.ds(0, D_NODE)], sem_a)
            g0d = pltpu.async_copy(nf_hbm.at[idxd.at[c0]], ra.at[:, pl.ds(D_NODE, D_NODE)], sem_a)
            g1s = pltpu.async_copy(nf_hbm.at[idxs.at[c1]], rb.at[:, pl.ds(0, D_NODE)], sem_b)
            g1d = pltpu.async_copy(nf_hbm.at[idxd.at[c1]], rb.at[:, pl.ds(D_NODE, D_NODE)], sem_b)
            g0s.wait()
            g0d.wait()
            w0 = pltpu.async_copy(ra, out_hbm.at[rows(c0)], sem_w)
            g1s.wait()
            g1d.wait()
            w1 = pltpu.async_copy(rb, out_hbm.at[rows(c1)], sem_w)
            w0.wait()
            w1.wait()

        lax.fori_loop(0, NCG // 2, body, None)

        ct = NCG - 1
        gts = pltpu.async_copy(nf_hbm.at[idxs.at[ct]], ra.at[:, pl.ds(0, D_NODE)], sem_a)
        gtd = pltpu.async_copy(nf_hbm.at[idxd.at[ct]], ra.at[:, pl.ds(D_NODE, D_NODE)], sem_a)
        gts.wait()
        gtd.wait()
        wt = pltpu.async_copy(ra, out_hbm.at[rows(ct)], sem_w)
        wt.wait()

    return _sc_gather


BE = 2560
NBG = EG // BE


def _mlp_body(vij_ref, ea_t_ref, ew_t_ref,
              Wv_lo_ref, Wv_hi_ref, W1ea_e_ref, b1e_ref, W2e_ref, b2e_ref,
              W1ea_n_ref, b1n_ref, W2n_ref, b2n_ref, We_ref, Wn_ref,
              ea_new_t_ref, feats_ref):
    f32 = jnp.float32
    bf = jnp.bfloat16

    def dg(lhs, rhs, dims):
        return lax.dot_general(lhs, rhs, (dims, ((), ())),
                               preferred_element_type=f32)

    v_lo = vij_ref[:, 0:D_NODE].astype(bf)
    v_hi = vij_ref[:, D_NODE:2 * D_NODE].astype(bf)
    ea_t = ea_t_ref[...]
    ea_t_bf = ea_t.astype(bf)
    ew_t = ew_t_ref[...].astype(bf)

    pre1 = (jnp.dot(v_lo, Wv_lo_ref[...], preferred_element_type=f32)
            + jnp.dot(v_hi, Wv_hi_ref[...], preferred_element_type=f32))

    he = pre1[:, 0:128] + dg(ea_t_bf, W1ea_e_ref[...], ((0,), (0,)))
    he = he + b1e_ref[...]
    he = he * jax.nn.sigmoid(he)
    s2e_t = dg(W2e_ref[...], he.astype(bf), ((0,), (1,))) + b2e_ref[...]
    ue_t = s2e_t[0:64]
    ue_t = ue_t * jax.nn.sigmoid(ue_t)
    ge_t = jax.nn.sigmoid(s2e_t[64:128])
    ewe_t = dg(We_ref[...], ew_t, ((0,), (0,)))
    ea_new_t = ea_t + ue_t * ge_t * ewe_t
    ea_new_t_ref[...] = ea_new_t

    hn = pre1[:, 128:256] + dg(ea_new_t.astype(bf), W1ea_n_ref[...], ((0,), (0,)))
    hn = hn + b1n_ref[...]
    hn = hn * jax.nn.sigmoid(hn)
    s2n = jnp.dot(hn.astype(bf), W2n_ref[...], preferred_element_type=f32) + b2n_ref[...]
    un = s2n[:, 0:128]
    un = un * jax.nn.sigmoid(un)
    gn = jax.nn.sigmoid(s2n[:, 128:256])
    ewn = dg(ew_t, Wn_ref[...], ((0,), (0,)))
    feats_ref[...] = un * gn * ewn


def _run_mlp(g, vij, ea_t, ew_t, Wv_lo, Wv_hi, W1ea_e, b1e, W2e, b2e, W1ea_n,
             b1n, W2n, b2n, We, Wn):
    blk = lambda shape: pl.BlockSpec(shape, lambda i: (0,) * len(shape))
    gbs = lambda w: pl.BlockSpec((BE, w), lambda i: (i, 0))
    tgbs = pl.BlockSpec((D_EDGE, BE), lambda i: (0, i))
    tfbs = pl.BlockSpec((D_EDGE, BE), lambda i: (0, g * NBG + i))
    return pl.pallas_call(
        _mlp_body,
        grid=(NBG,),
        in_specs=[
            gbs(256), tfbs, tfbs,
            blk((128, 256)), blk((128, 256)), blk((64, 128)), blk((1, 128)),
            blk((128, 128)), blk((128, 1)), blk((64, 128)), blk((1, 128)),
            blk((128, 256)), blk((1, 256)), blk((64, 64)), blk((64, 128)),
        ],
        out_specs=[tgbs, gbs(128)],
        out_shape=[
            jax.ShapeDtypeStruct((D_EDGE, EG), jnp.float32),
            jax.ShapeDtypeStruct((EG, D_NODE), jnp.float32),
        ],
    )(vij, ea_t, ew_t, Wv_lo, Wv_hi, W1ea_e, b1e, W2e, b2e, W1ea_n, b1n, W2n,
      b2n, We, Wn)


@functools.lru_cache(maxsize=None)
def _get_sc_scatter(ng, seed_nf):
    @functools.partial(
        pl.kernel,
        out_type=jax.ShapeDtypeStruct((NC, N_NODES, D_NODE), jnp.float32),
        mesh=_get_sc_mesh(),
        scratch_types=[
            pltpu.VMEM_SHARED((N_NODES, D_NODE), jnp.float32),
            pltpu.VMEM((NCG, CHUNK), jnp.int32),
            pltpu.VMEM((CHUNK, D_NODE), jnp.float32),
            pltpu.VMEM((CHUNK, D_NODE), jnp.float32),
            pltpu.SemaphoreType.DMA,
            pltpu.SemaphoreType.DMA,
        ],
    )
    def _sc_scatter(*args):
        feats_list = args[0:ng]
        srcm_hbm, nf_hbm, zeros_hbm, out_hbm = args[ng:ng + 4]
        acc, idxs, rowa, rowb, sem_a, sem_b = args[ng + 4:]
        cid = lax.axis_index("c")
        sid = lax.axis_index("s")
        wid = sid * NC + cid
        base = wid * E_PER_WG
        roff = sid * ROWS_PER_TILE

        @pl.when(cid == 0)
        def _():
            seed_hbm = nf_hbm if seed_nf else zeros_hbm
            pltpu.sync_copy(seed_hbm.at[pl.ds(roff, ROWS_PER_TILE)],
                            acc.at[pl.ds(roff, ROWS_PER_TILE)])

            @pl.when(sid == NS - 1)
            def _():
                pltpu.sync_copy(seed_hbm.at[pl.ds(TAIL_OFF + ROWS_PER_TILE, TAIL_ROWS - ROWS_PER_TILE)],
                                acc.at[pl.ds(TAIL_OFF + ROWS_PER_TILE, TAIL_ROWS - ROWS_PER_TILE)])

        @pl.when(cid != 0)
        def _():
            pltpu.sync_copy(zeros_hbm.at[pl.ds(roff, ROWS_PER_TILE)],
                            acc.at[pl.ds(roff, ROWS_PER_TILE)])

            @pl.when(sid == NS - 1)
            def _():
                pltpu.sync_copy(zeros_hbm.at[pl.ds(TAIL_OFF + ROWS_PER_TILE, TAIL_ROWS - ROWS_PER_TILE)],
                                acc.at[pl.ds(TAIL_OFF + ROWS_PER_TILE, TAIL_ROWS - ROWS_PER_TILE)])

        plsc.subcore_barrier()

        def rows(c):
            return pl.ds(base + c * CHUNK, CHUNK)

        for gi, feats_hbm in enumerate(feats_list):
            pltpu.sync_copy(srcm_hbm.at[gi, wid], idxs)

            def body(i, _, feats_hbm=feats_hbm):
                c0 = 2 * i
                c1 = 2 * i + 1
                fa = pltpu.async_copy(feats_hbm.at[rows(c0)], rowa, sem_a)
                fb = pltpu.async_copy(feats_hbm.at[rows(c1)], rowb, sem_b)
                fa.wait()
                pltpu.sync_copy(rowa, acc.at[idxs.at[c0]], add=True)
                fb.wait()
                pltpu.sync_copy(rowb, acc.at[idxs.at[c1]], add=True)

            lax.fori_loop(0, NCG // 2, body, None)

            ct = NCG - 1
            ft = pltpu.async_copy(feats_hbm.at[rows(ct)], rowa, sem_a)
            ft.wait()
            pltpu.sync_copy(rowa, acc.at[idxs.at[ct]], add=True)

        plsc.subcore_barrier()
        pltpu.sync_copy(acc.at[pl.ds(roff, ROWS_PER_TILE)],
                        out_hbm.at[cid, pl.ds(roff, ROWS_PER_TILE)])

        @pl.when(sid == NS - 1)
        def _():
            pltpu.sync_copy(acc.at[pl.ds(TAIL_OFF + ROWS_PER_TILE, TAIL_ROWS - ROWS_PER_TILE)],
                            out_hbm.at[cid, pl.ds(TAIL_OFF + ROWS_PER_TILE, TAIL_ROWS - ROWS_PER_TILE)])

    return _sc_scatter


def _combine_body(p_ref, q_ref, out_ref):
    out_ref[...] = (p_ref[0] + p_ref[1]) + (q_ref[0] + q_ref[1])


def _run_combine(parts_a, parts_b):
    nb = 10
    rb = N_NODES // nb
    spec = pl.BlockSpec((NC, rb, D_NODE), lambda i: (0, i, 0))
    return pl.pallas_call(
        _combine_body,
        grid=(nb,),
        in_specs=[spec, spec],
        out_specs=pl.BlockSpec((rb, D_NODE), lambda i: (i, 0)),
        out_shape=jax.ShapeDtypeStruct((N_NODES, D_NODE), jnp.float32),
    )(parts_a, parts_b)


def kernel(node_features, edge_index, edge_attr, edge_weights,
           eW1, eb1, eW2, eb2, egW1, egb1, egW2, egb2,
           nW1, nb1, nW2, nb2, ngW1, ngb1, ngW2, ngb2,
           We, Wn):
    bf = jnp.bfloat16
    src = edge_index[0].astype(jnp.int32)
    dst = edge_index[1].astype(jnp.int32)
    srcm = src.reshape(G, NW, NCG, CHUNK)
    dstm = dst.reshape(G, NW, NCG, CHUNK)

    top = jnp.concatenate([eW1[0:128], egW1[0:128], nW1[0:128], ngW1[0:128]], axis=1)
    bot = jnp.concatenate([eW1[128:256], egW1[128:256], nW1[128:256], ngW1[128:256]], axis=1)
    Wv_lo = top.astype(bf)
    Wv_hi = bot.astype(bf)
    W1ea_e = jnp.concatenate([eW1[256:320], egW1[256:320]], axis=1).astype(bf)
    W1ea_n = jnp.concatenate([nW1[256:320], ngW1[256:320]], axis=1).astype(bf)
    b1e = jnp.concatenate([eb1, egb1])[None, :]
    b1n = jnp.concatenate([nb1, ngb1])[None, :]
    z64 = jnp.zeros((64, 64), jnp.float32)
    W2e = jnp.block([[eW2, z64], [z64, egW2]]).astype(bf)
    b2e = jnp.concatenate([eb2, egb2])[:, None]
    z64n = jnp.zeros((64, 128), jnp.float32)
    W2n = jnp.block([[nW2, z64n], [z64n, ngW2]]).astype(bf)
    b2n = jnp.concatenate([nb2, ngb2])[None, :]
    We_bf = We.astype(bf)
    Wn_bf = Wn.astype(bf)

    ea_t = edge_attr.T
    ew_t = edge_weights.T

    gather = _get_sc_gather()
    ean_t_gs = []
    feats_gs = []
    for g in range(G):
        vij_g = gather(node_features, srcm[g], dstm[g])
        ean_t_g, feats_g = _run_mlp(g, vij_g, ea_t, ew_t,
                                    Wv_lo, Wv_hi, W1ea_e, b1e, W2e, b2e,
                                    W1ea_n, b1n, W2n, b2n, We_bf, Wn_bf)
        ean_t_gs.append(ean_t_g)
        feats_gs.append(feats_g)

    ea_new = jnp.concatenate(ean_t_gs, axis=1).T
    zeros = jnp.zeros((N_NODES, D_NODE), jnp.float32)
    parts_a = _get_sc_scatter(3, True)(*feats_gs[0:3], srcm[0:3],
                                       node_features, zeros)
    parts_b = _get_sc_scatter(2, False)(*feats_gs[3:5], srcm[3:5],
                                        node_features, zeros)
    node_new = _run_combine(parts_a, parts_b)
    return (node_new, ea_new)


# speedup vs baseline: 1.6314x; 1.0542x over previous
import functools

import jax
import jax.numpy as jnp
from jax import lax
from jax.experimental import pallas as pl
from jax.experimental.pallas import tpu as pltpu
from jax.experimental.pallas import tpu_sc as plsc

N_NODES = 10000
N_EDGES = 320000
D_NODE = 128
D_EDGE = 64
DEGREE = 64

NC = 2
NS = 16
NW = NC * NS
G = 5
EG = N_EDGES // G
E_PER_WG = EG // NW
CHUNK = 80
NCG = E_PER_WG // CHUNK
ROWS_PER_TILE = 624
TAIL_OFF = 15 * ROWS_PER_TILE
TAIL_ROWS = N_NODES - TAIL_OFF


@functools.lru_cache(maxsize=None)
def _get_sc_mesh():
    return plsc.VectorSubcoreMesh(core_axis_name="c", subcore_axis_name="s")


@functools.lru_cache(maxsize=None)
def _get_sc_gather():
    @functools.partial(
        pl.kernel,
        out_type=jax.ShapeDtypeStruct((EG, 2 * D_NODE), jnp.float32),
        mesh=_get_sc_mesh(),
        scratch_types=[
            pltpu.VMEM((NCG, CHUNK), jnp.int32),
            pltpu.VMEM((NCG, CHUNK), jnp.int32),
            pltpu.VMEM((CHUNK, 2 * D_NODE), jnp.float32),
            pltpu.VMEM((CHUNK, 2 * D_NODE), jnp.float32),
            pltpu.SemaphoreType.DMA,
            pltpu.SemaphoreType.DMA,
            pltpu.SemaphoreType.DMA,
        ],
    )
    def _sc_gather(nf_hbm, srcm_hbm, dstm_hbm, out_hbm,
                   idxs, idxd, ra, rb, sem_a, sem_b, sem_w):
        wid = lax.axis_index("s") * NC + lax.axis_index("c")
        base = wid * E_PER_WG

        pltpu.sync_copy(srcm_hbm.at[wid], idxs)
        pltpu.sync_copy(dstm_hbm.at[wid], idxd)

        def rows(c):
            return pl.ds(base + c * CHUNK, CHUNK)

        def body(i, _):
            c0 = 2 * i
            c1 = 2 * i + 1
            g0s = pltpu.async_copy(nf_hbm.at[idxs.at[c0]], ra.at[:, pl.ds(0, D_NODE)], sem_a)
            g0d = pltpu.async_copy(nf_hbm.at[idxd.at[c0]], ra.at[:, pl.ds(D_NODE, D_NODE)], sem_a)
            g1s = pltpu.async_copy(nf_hbm.at[idxs.at[c1]], rb.at[:, pl.ds(0, D_NODE)], sem_b)
            g1d = pltpu.async_copy(nf_hbm.at[idxd.at[c1]], rb.at[:, pl.ds(D_NODE, D_NODE)], sem_b)
            g0s.wait()
            g0d.wait()
            w0 = pltpu.async_copy(ra, out_hbm.at[rows(c0)], sem_w)
            g1s.wait()
            g1d.wait()
            w1 = pltpu.async_copy(rb, out_hbm.at[rows(c1)], sem_w)
            w0.wait()
            w1.wait()

        lax.fori_loop(0, NCG // 2, body, None)

        ct = NCG - 1
        gts = pltpu.async_copy(nf_hbm.at[idxs.at[ct]], ra.at[:, pl.ds(0, D_NODE)], sem_a)
        gtd = pltpu.async_copy(nf_hbm.at[idxd.at[ct]], ra.at[:, pl.ds(D_NODE, D_NODE)], sem_a)
        gts.wait()
        gtd.wait()
        wt = pltpu.async_copy(ra, out_hbm.at[rows(ct)], sem_w)
        wt.wait()

    return _sc_gather


BE = 3200
NBG = EG // BE


def _mlp_body(vij_ref, ea_t_ref, ew_t_ref,
              Wv_lo_ref, Wv_hi_ref, W1ea_e_ref, b1e_ref, W2e_ref, b2e_ref,
              W1ea_n_ref, b1n_ref, W2n_ref, b2n_ref, We_ref, Wn_ref,
              ea_new_t_ref, feats_ref):
    f32 = jnp.float32
    bf = jnp.bfloat16

    def dg(lhs, rhs, dims):
        return lax.dot_general(lhs, rhs, (dims, ((), ())),
                               preferred_element_type=f32)

    v_lo = vij_ref[:, 0:D_NODE].astype(bf)
    v_hi = vij_ref[:, D_NODE:2 * D_NODE].astype(bf)
    ea_t = ea_t_ref[...]
    ea_t_bf = ea_t.astype(bf)
    ew_t = ew_t_ref[...].astype(bf)

    pre1 = (jnp.dot(v_lo, Wv_lo_ref[...], preferred_element_type=f32)
            + jnp.dot(v_hi, Wv_hi_ref[...], preferred_element_type=f32))

    he = pre1[:, 0:128] + dg(ea_t_bf, W1ea_e_ref[...], ((0,), (0,)))
    he = he + b1e_ref[...]
    he = he * jax.nn.sigmoid(he)
    s2e_t = dg(W2e_ref[...], he.astype(bf), ((0,), (1,))) + b2e_ref[...]
    ue_t = s2e_t[0:64]
    ue_t = ue_t * jax.nn.sigmoid(ue_t)
    ge_t = jax.nn.sigmoid(s2e_t[64:128])
    ewe_t = dg(We_ref[...], ew_t, ((0,), (0,)))
    ea_new_t = ea_t + ue_t * ge_t * ewe_t
    ea_new_t_ref[...] = ea_new_t

    hn = pre1[:, 128:256] + dg(ea_new_t.astype(bf), W1ea_n_ref[...], ((0,), (0,)))
    hn = hn + b1n_ref[...]
    hn = hn * jax.nn.sigmoid(hn)
    s2n = jnp.dot(hn.astype(bf), W2n_ref[...], preferred_element_type=f32) + b2n_ref[...]
    un = s2n[:, 0:128]
    un = un * jax.nn.sigmoid(un)
    gn = jax.nn.sigmoid(s2n[:, 128:256])
    ewn = dg(ew_t, Wn_ref[...], ((0,), (0,)))
    feats_ref[...] = un * gn * ewn


def _run_mlp(g, vij, ea_t, ew_t, Wv_lo, Wv_hi, W1ea_e, b1e, W2e, b2e, W1ea_n,
             b1n, W2n, b2n, We, Wn):
    blk = lambda shape: pl.BlockSpec(shape, lambda i: (0,) * len(shape))
    gbs = lambda w: pl.BlockSpec((BE, w), lambda i: (i, 0))
    tgbs = pl.BlockSpec((D_EDGE, BE), lambda i: (0, i))
    tfbs = pl.BlockSpec((D_EDGE, BE), lambda i: (0, g * NBG + i))
    return pl.pallas_call(
        _mlp_body,
        grid=(NBG,),
        in_specs=[
            gbs(256), tfbs, tfbs,
            blk((128, 256)), blk((128, 256)), blk((64, 128)), blk((1, 128)),
            blk((128, 128)), blk((128, 1)), blk((64, 128)), blk((1, 128)),
            blk((128, 256)), blk((1, 256)), blk((64, 64)), blk((64, 128)),
        ],
        out_specs=[tgbs, gbs(128)],
        out_shape=[
            jax.ShapeDtypeStruct((D_EDGE, EG), jnp.float32),
            jax.ShapeDtypeStruct((EG, D_NODE), jnp.float32),
        ],
    )(vij, ea_t, ew_t, Wv_lo, Wv_hi, W1ea_e, b1e, W2e, b2e, W1ea_n, b1n, W2n,
      b2n, We, Wn)


@functools.lru_cache(maxsize=None)
def _get_sc_scatter(ng, seed_nf):
    @functools.partial(
        pl.kernel,
        out_type=jax.ShapeDtypeStruct((NC, N_NODES, D_NODE), jnp.float32),
        mesh=_get_sc_mesh(),
        scratch_types=[
            pltpu.VMEM_SHARED((N_NODES, D_NODE), jnp.float32),
            pltpu.VMEM((NCG, CHUNK), jnp.int32),
            pltpu.VMEM((CHUNK, D_NODE), jnp.float32),
            pltpu.VMEM((CHUNK, D_NODE), jnp.float32),
            pltpu.SemaphoreType.DMA,
            pltpu.SemaphoreType.DMA,
        ],
    )
    def _sc_scatter(*args):
        feats_list = args[0:ng]
        srcm_hbm, nf_hbm, zeros_hbm, out_hbm = args[ng:ng + 4]
        acc, idxs, rowa, rowb, sem_a, sem_b = args[ng + 4:]
        cid = lax.axis_index("c")
        sid = lax.axis_index("s")
        wid = sid * NC + cid
        base = wid * E_PER_WG
        roff = sid * ROWS_PER_TILE

        @pl.when(cid == 0)
        def _():
            seed_hbm = nf_hbm if seed_nf else zeros_hbm
            pltpu.sync_copy(seed_hbm.at[pl.ds(roff, ROWS_PER_TILE)],
                            acc.at[pl.ds(roff, ROWS_PER_TILE)])

            @pl.when(sid == NS - 1)
            def _():
                pltpu.sync_copy(seed_hbm.at[pl.ds(TAIL_OFF + ROWS_PER_TILE, TAIL_ROWS - ROWS_PER_TILE)],
                                acc.at[pl.ds(TAIL_OFF + ROWS_PER_TILE, TAIL_ROWS - ROWS_PER_TILE)])

        @pl.when(cid != 0)
        def _():
            pltpu.sync_copy(zeros_hbm.at[pl.ds(roff, ROWS_PER_TILE)],
                            acc.at[pl.ds(roff, ROWS_PER_TILE)])

            @pl.when(sid == NS - 1)
            def _():
                pltpu.sync_copy(zeros_hbm.at[pl.ds(TAIL_OFF + ROWS_PER_TILE, TAIL_ROWS - ROWS_PER_TILE)],
                                acc.at[pl.ds(TAIL_OFF + ROWS_PER_TILE, TAIL_ROWS - ROWS_PER_TILE)])

        plsc.subcore_barrier()

        def rows(c):
            return pl.ds(base + c * CHUNK, CHUNK)

        for gi, feats_hbm in enumerate(feats_list):
            pltpu.sync_copy(srcm_hbm.at[gi, wid], idxs)

            def body(i, _, feats_hbm=feats_hbm):
                c0 = 2 * i
                c1 = 2 * i + 1
                fa = pltpu.async_copy(feats_hbm.at[rows(c0)], rowa, sem_a)
                fb = pltpu.async_copy(feats_hbm.at[rows(c1)], rowb, sem_b)
                fa.wait()
                pltpu.sync_copy(rowa, acc.at[idxs.at[c0]], add=True)
                fb.wait()
                pltpu.sync_copy(rowb, acc.at[idxs.at[c1]], add=True)

            lax.fori_loop(0, NCG // 2, body, None)

            ct = NCG - 1
            ft = pltpu.async_copy(feats_hbm.at[rows(ct)], rowa, sem_a)
            ft.wait()
            pltpu.sync_copy(rowa, acc.at[idxs.at[ct]], add=True)

        plsc.subcore_barrier()
        pltpu.sync_copy(acc.at[pl.ds(roff, ROWS_PER_TILE)],
                        out_hbm.at[cid, pl.ds(roff, ROWS_PER_TILE)])

        @pl.when(sid == NS - 1)
        def _():
            pltpu.sync_copy(acc.at[pl.ds(TAIL_OFF + ROWS_PER_TILE, TAIL_ROWS - ROWS_PER_TILE)],
                            out_hbm.at[cid, pl.ds(TAIL_OFF + ROWS_PER_TILE, TAIL_ROWS - ROWS_PER_TILE)])

    return _sc_scatter


def _combine_body(p_ref, q_ref, r_ref, out_ref):
    out_ref[...] = ((p_ref[0] + p_ref[1]) + (q_ref[0] + q_ref[1])
                    + (r_ref[0] + r_ref[1]))


def _run_combine(parts_a, parts_b, parts_c):
    nb = 10
    rb = N_NODES // nb
    spec = pl.BlockSpec((NC, rb, D_NODE), lambda i: (0, i, 0))
    return pl.pallas_call(
        _combine_body,
        grid=(nb,),
        in_specs=[spec, spec, spec],
        out_specs=pl.BlockSpec((rb, D_NODE), lambda i: (i, 0)),
        out_shape=jax.ShapeDtypeStruct((N_NODES, D_NODE), jnp.float32),
    )(parts_a, parts_b, parts_c)


def kernel(node_features, edge_index, edge_attr, edge_weights,
           eW1, eb1, eW2, eb2, egW1, egb1, egW2, egb2,
           nW1, nb1, nW2, nb2, ngW1, ngb1, ngW2, ngb2,
           We, Wn):
    bf = jnp.bfloat16
    src = edge_index[0].astype(jnp.int32)
    dst = edge_index[1].astype(jnp.int32)
    srcm = src.reshape(G, NW, NCG, CHUNK)
    dstm = dst.reshape(G, NW, NCG, CHUNK)

    top = jnp.concatenate([eW1[0:128], egW1[0:128], nW1[0:128], ngW1[0:128]], axis=1)
    bot = jnp.concatenate([eW1[128:256], egW1[128:256], nW1[128:256], ngW1[128:256]], axis=1)
    Wv_lo = top.astype(bf)
    Wv_hi = bot.astype(bf)
    W1ea_e = jnp.concatenate([eW1[256:320], egW1[256:320]], axis=1).astype(bf)
    W1ea_n = jnp.concatenate([nW1[256:320], ngW1[256:320]], axis=1).astype(bf)
    b1e = jnp.concatenate([eb1, egb1])[None, :]
    b1n = jnp.concatenate([nb1, ngb1])[None, :]
    z64 = jnp.zeros((64, 64), jnp.float32)
    W2e = jnp.block([[eW2, z64], [z64, egW2]]).astype(bf)
    b2e = jnp.concatenate([eb2, egb2])[:, None]
    z64n = jnp.zeros((64, 128), jnp.float32)
    W2n = jnp.block([[nW2, z64n], [z64n, ngW2]]).astype(bf)
    b2n = jnp.concatenate([nb2, ngb2])[None, :]
    We_bf = We.astype(bf)
    Wn_bf = Wn.astype(bf)

    ea_t = edge_attr.T
    ew_t = edge_weights.T

    gather = _get_sc_gather()
    ean_t_gs = []
    feats_gs = []
    for g in range(G):
        vij_g = gather(node_features, srcm[g], dstm[g])
        ean_t_g, feats_g = _run_mlp(g, vij_g, ea_t, ew_t,
                                    Wv_lo, Wv_hi, W1ea_e, b1e, W2e, b2e,
                                    W1ea_n, b1n, W2n, b2n, We_bf, Wn_bf)
        ean_t_gs.append(ean_t_g)
        feats_gs.append(feats_g)

    ea_new = jnp.concatenate(ean_t_gs, axis=1).T
    zeros = jnp.zeros((N_NODES, D_NODE), jnp.float32)
    parts_a = _get_sc_scatter(2, True)(*feats_gs[0:2], srcm[0:2],
                                       node_features, zeros)
    parts_b = _get_sc_scatter(2, False)(*feats_gs[2:4], srcm[2:4],
                                        node_features, zeros)
    parts_c = _get_sc_scatter(1, False)(*feats_gs[4:5], srcm[4:5],
                                        node_features, zeros)
    node_new = _run_combine(parts_a, parts_b, parts_c)
    return (node_new, ea_new)


# speedup vs baseline: 1.6404x; 1.0055x over previous
import functools

import jax
import jax.numpy as jnp
from jax import lax
from jax.experimental import pallas as pl
from jax.experimental.pallas import tpu as pltpu
from jax.experimental.pallas import tpu_sc as plsc

N_NODES = 10000
N_EDGES = 320000
D_NODE = 128
D_EDGE = 64
DEGREE = 64

NC = 2
NS = 16
NW = NC * NS
G = 5
EG = N_EDGES // G
E_PER_WG = EG // NW
CHUNK = 80
NCG = E_PER_WG // CHUNK
ROWS_PER_TILE = 624
TAIL_OFF = 15 * ROWS_PER_TILE
TAIL_ROWS = N_NODES - TAIL_OFF


@functools.lru_cache(maxsize=None)
def _get_sc_mesh():
    return plsc.VectorSubcoreMesh(core_axis_name="c", subcore_axis_name="s")


@functools.lru_cache(maxsize=None)
def _get_sc_gather():
    @functools.partial(
        pl.kernel,
        out_type=jax.ShapeDtypeStruct((EG, 2 * D_NODE), jnp.float32),
        mesh=_get_sc_mesh(),
        scratch_types=[
            pltpu.VMEM((NCG, CHUNK), jnp.int32),
            pltpu.VMEM((NCG, CHUNK), jnp.int32),
            pltpu.VMEM((CHUNK, 2 * D_NODE), jnp.float32),
            pltpu.VMEM((CHUNK, 2 * D_NODE), jnp.float32),
            pltpu.SemaphoreType.DMA,
            pltpu.SemaphoreType.DMA,
            pltpu.SemaphoreType.DMA,
        ],
    )
    def _sc_gather(nf_hbm, srcm_hbm, dstm_hbm, out_hbm,
                   idxs, idxd, ra, rb, sem_a, sem_b, sem_w):
        wid = lax.axis_index("s") * NC + lax.axis_index("c")
        base = wid * E_PER_WG

        pltpu.sync_copy(srcm_hbm.at[wid], idxs)
        pltpu.sync_copy(dstm_hbm.at[wid], idxd)

        def rows(c):
            return pl.ds(base + c * CHUNK, CHUNK)

        def body(i, _):
            c0 = 2 * i
            c1 = 2 * i + 1
            g0s = pltpu.async_copy(nf_hbm.at[idxs.at[c0]], ra.at[:, pl.ds(0, D_NODE)], sem_a)
            g0d = pltpu.async_copy(nf_hbm.at[idxd.at[c0]], ra.at[:, pl.ds(D_NODE, D_NODE)], sem_a)
            g1s = pltpu.async_copy(nf_hbm.at[idxs.at[c1]], rb.at[:, pl.ds(0, D_NODE)], sem_b)
            g1d = pltpu.async_copy(nf_hbm.at[idxd.at[c1]], rb.at[:, pl.ds(D_NODE, D_NODE)], sem_b)
            g0s.wait()
            g0d.wait()
            w0 = pltpu.async_copy(ra, out_hbm.at[rows(c0)], sem_w)
            g1s.wait()
            g1d.wait()
            w1 = pltpu.async_copy(rb, out_hbm.at[rows(c1)], sem_w)
            w0.wait()
            w1.wait()

        lax.fori_loop(0, NCG // 2, body, None)

        ct = NCG - 1
        gts = pltpu.async_copy(nf_hbm.at[idxs.at[ct]], ra.at[:, pl.ds(0, D_NODE)], sem_a)
        gtd = pltpu.async_copy(nf_hbm.at[idxd.at[ct]], ra.at[:, pl.ds(D_NODE, D_NODE)], sem_a)
        gts.wait()
        gtd.wait()
        wt = pltpu.async_copy(ra, out_hbm.at[rows(ct)], sem_w)
        wt.wait()

    return _sc_gather


BE = 6400
NBG = EG // BE


def _sig(x):
    return 0.5 * jnp.tanh(0.5 * x) + 0.5


def _mlp_body(vij_ref, ea_t_ref, ew_t_ref,
              Wv_lo_ref, Wv_hi_ref, W1ea_e_ref, b1e_ref, W2e_ref, b2e_ref,
              W1ea_n_ref, b1n_ref, W2n_ref, b2n_ref, We_ref, Wn_ref,
              ea_new_t_ref, feats_ref):
    f32 = jnp.float32
    bf = jnp.bfloat16

    def dg(lhs, rhs, dims):
        return lax.dot_general(lhs, rhs, (dims, ((), ())),
                               preferred_element_type=f32)

    v_lo = vij_ref[:, 0:D_NODE].astype(bf)
    v_hi = vij_ref[:, D_NODE:2 * D_NODE].astype(bf)
    ea_t = ea_t_ref[...]
    ea_t_bf = ea_t.astype(bf)
    ew_t = ew_t_ref[...].astype(bf)

    pre1 = (jnp.dot(v_lo, Wv_lo_ref[...], preferred_element_type=f32)
            + jnp.dot(v_hi, Wv_hi_ref[...], preferred_element_type=f32))

    he = pre1[:, 0:128] + dg(ea_t_bf, W1ea_e_ref[...], ((0,), (0,)))
    he = he + b1e_ref[...]
    he = he * _sig(he)
    s2e_t = dg(W2e_ref[...], he.astype(bf), ((0,), (1,))) + b2e_ref[...]
    ue_t = s2e_t[0:64]
    ue_t = ue_t * _sig(ue_t)
    ge_t = _sig(s2e_t[64:128])
    ewe_t = dg(We_ref[...], ew_t, ((0,), (0,)))
    ea_new_t = ea_t + ue_t * ge_t * ewe_t
    ea_new_t_ref[...] = ea_new_t

    hn = pre1[:, 128:256] + dg(ea_new_t.astype(bf), W1ea_n_ref[...], ((0,), (0,)))
    hn = hn + b1n_ref[...]
    hn = hn * _sig(hn)
    s2n = jnp.dot(hn.astype(bf), W2n_ref[...], preferred_element_type=f32) + b2n_ref[...]
    un = s2n[:, 0:128]
    un = un * _sig(un)
    gn = _sig(s2n[:, 128:256])
    ewn = dg(ew_t, Wn_ref[...], ((0,), (0,)))
    feats_ref[...] = un * gn * ewn


def _run_mlp(g, vij, ea_t, ew_t, Wv_lo, Wv_hi, W1ea_e, b1e, W2e, b2e, W1ea_n,
             b1n, W2n, b2n, We, Wn):
    blk = lambda shape: pl.BlockSpec(shape, lambda i: (0,) * len(shape))
    gbs = lambda w: pl.BlockSpec((BE, w), lambda i: (i, 0))
    tgbs = pl.BlockSpec((D_EDGE, BE), lambda i: (0, i))
    tfbs = pl.BlockSpec((D_EDGE, BE), lambda i: (0, g * NBG + i))
    return pl.pallas_call(
        _mlp_body,
        grid=(NBG,),
        in_specs=[
            gbs(256), tfbs, tfbs,
            blk((128, 256)), blk((128, 256)), blk((64, 128)), blk((1, 128)),
            blk((128, 128)), blk((128, 1)), blk((64, 128)), blk((1, 128)),
            blk((128, 256)), blk((1, 256)), blk((64, 64)), blk((64, 128)),
        ],
        out_specs=[tgbs, gbs(128)],
        out_shape=[
            jax.ShapeDtypeStruct((D_EDGE, EG), jnp.float32),
            jax.ShapeDtypeStruct((EG, D_NODE), jnp.float32),
        ],
    )(vij, ea_t, ew_t, Wv_lo, Wv_hi, W1ea_e, b1e, W2e, b2e, W1ea_n, b1n, W2n,
      b2n, We, Wn)


@functools.lru_cache(maxsize=None)
def _get_sc_scatter(ng, seed_nf):
    @functools.partial(
        pl.kernel,
        out_type=jax.ShapeDtypeStruct((NC, N_NODES, D_NODE), jnp.float32),
        mesh=_get_sc_mesh(),
        scratch_types=[
            pltpu.VMEM_SHARED((N_NODES, D_NODE), jnp.float32),
            pltpu.VMEM((NCG, CHUNK), jnp.int32),
            pltpu.VMEM((CHUNK, D_NODE), jnp.float32),
            pltpu.VMEM((CHUNK, D_NODE), jnp.float32),
            pltpu.SemaphoreType.DMA,
            pltpu.SemaphoreType.DMA,
        ],
    )
    def _sc_scatter(*args):
        feats_list = args[0:ng]
        srcm_hbm, nf_hbm, zeros_hbm, out_hbm = args[ng:ng + 4]
        acc, idxs, rowa, rowb, sem_a, sem_b = args[ng + 4:]
        cid = lax.axis_index("c")
        sid = lax.axis_index("s")
        wid = sid * NC + cid
        base = wid * E_PER_WG
        roff = sid * ROWS_PER_TILE

        @pl.when(cid == 0)
        def _():
            seed_hbm = nf_hbm if seed_nf else zeros_hbm
            pltpu.sync_copy(seed_hbm.at[pl.ds(roff, ROWS_PER_TILE)],
                            acc.at[pl.ds(roff, ROWS_PER_TILE)])

            @pl.when(sid == NS - 1)
            def _():
                pltpu.sync_copy(seed_hbm.at[pl.ds(TAIL_OFF + ROWS_PER_TILE, TAIL_ROWS - ROWS_PER_TILE)],
                                acc.at[pl.ds(TAIL_OFF + ROWS_PER_TILE, TAIL_ROWS - ROWS_PER_TILE)])

        @pl.when(cid != 0)
        def _():
            pltpu.sync_copy(zeros_hbm.at[pl.ds(roff, ROWS_PER_TILE)],
                            acc.at[pl.ds(roff, ROWS_PER_TILE)])

            @pl.when(sid == NS - 1)
            def _():
                pltpu.sync_copy(zeros_hbm.at[pl.ds(TAIL_OFF + ROWS_PER_TILE, TAIL_ROWS - ROWS_PER_TILE)],
                                acc.at[pl.ds(TAIL_OFF + ROWS_PER_TILE, TAIL_ROWS - ROWS_PER_TILE)])

        plsc.subcore_barrier()

        def rows(c):
            return pl.ds(base + c * CHUNK, CHUNK)

        for gi, feats_hbm in enumerate(feats_list):
            pltpu.sync_copy(srcm_hbm.at[gi, wid], idxs)

            def body(i, _, feats_hbm=feats_hbm):
                c0 = 2 * i
                c1 = 2 * i + 1
                fa = pltpu.async_copy(feats_hbm.at[rows(c0)], rowa, sem_a)
                fb = pltpu.async_copy(feats_hbm.at[rows(c1)], rowb, sem_b)
                fa.wait()
                pltpu.sync_copy(rowa, acc.at[idxs.at[c0]], add=True)
                fb.wait()
                pltpu.sync_copy(rowb, acc.at[idxs.at[c1]], add=True)

            lax.fori_loop(0, NCG // 2, body, None)

            ct = NCG - 1
            ft = pltpu.async_copy(feats_hbm.at[rows(ct)], rowa, sem_a)
            ft.wait()
            pltpu.sync_copy(rowa, acc.at[idxs.at[ct]], add=True)

        plsc.subcore_barrier()
        pltpu.sync_copy(acc.at[pl.ds(roff, ROWS_PER_TILE)],
                        out_hbm.at[cid, pl.ds(roff, ROWS_PER_TILE)])

        @pl.when(sid == NS - 1)
        def _():
            pltpu.sync_copy(acc.at[pl.ds(TAIL_OFF + ROWS_PER_TILE, TAIL_ROWS - ROWS_PER_TILE)],
                            out_hbm.at[cid, pl.ds(TAIL_OFF + ROWS_PER_TILE, TAIL_ROWS - ROWS_PER_TILE)])

    return _sc_scatter


def _combine_body(p_ref, q_ref, r_ref, out_ref):
    out_ref[...] = ((p_ref[0] + p_ref[1]) + (q_ref[0] + q_ref[1])
                    + (r_ref[0] + r_ref[1]))


def _run_combine(parts_a, parts_b, parts_c):
    nb = 10
    rb = N_NODES // nb
    spec = pl.BlockSpec((NC, rb, D_NODE), lambda i: (0, i, 0))
    return pl.pallas_call(
        _combine_body,
        grid=(nb,),
        in_specs=[spec, spec, spec],
        out_specs=pl.BlockSpec((rb, D_NODE), lambda i: (i, 0)),
        out_shape=jax.ShapeDtypeStruct((N_NODES, D_NODE), jnp.float32),
    )(parts_a, parts_b, parts_c)


def kernel(node_features, edge_index, edge_attr, edge_weights,
           eW1, eb1, eW2, eb2, egW1, egb1, egW2, egb2,
           nW1, nb1, nW2, nb2, ngW1, ngb1, ngW2, ngb2,
           We, Wn):
    bf = jnp.bfloat16
    src = edge_index[0].astype(jnp.int32)
    dst = edge_index[1].astype(jnp.int32)
    srcm = src.reshape(G, NW, NCG, CHUNK)
    dstm = dst.reshape(G, NW, NCG, CHUNK)

    top = jnp.concatenate([eW1[0:128], egW1[0:128], nW1[0:128], ngW1[0:128]], axis=1)
    bot = jnp.concatenate([eW1[128:256], egW1[128:256], nW1[128:256], ngW1[128:256]], axis=1)
    Wv_lo = top.astype(bf)
    Wv_hi = bot.astype(bf)
    W1ea_e = jnp.concatenate([eW1[256:320], egW1[256:320]], axis=1).astype(bf)
    W1ea_n = jnp.concatenate([nW1[256:320], ngW1[256:320]], axis=1).astype(bf)
    b1e = jnp.concatenate([eb1, egb1])[None, :]
    b1n = jnp.concatenate([nb1, ngb1])[None, :]
    z64 = jnp.zeros((64, 64), jnp.float32)
    W2e = jnp.block([[eW2, z64], [z64, egW2]]).astype(bf)
    b2e = jnp.concatenate([eb2, egb2])[:, None]
    z64n = jnp.zeros((64, 128), jnp.float32)
    W2n = jnp.block([[nW2, z64n], [z64n, ngW2]]).astype(bf)
    b2n = jnp.concatenate([nb2, ngb2])[None, :]
    We_bf = We.astype(bf)
    Wn_bf = Wn.astype(bf)

    ea_t = edge_attr.T
    ew_t = edge_weights.T

    gather = _get_sc_gather()
    ean_t_gs = []
    feats_gs = []
    for g in range(G):
        vij_g = gather(node_features, srcm[g], dstm[g])
        ean_t_g, feats_g = _run_mlp(g, vij_g, ea_t, ew_t,
                                    Wv_lo, Wv_hi, W1ea_e, b1e, W2e, b2e,
                                    W1ea_n, b1n, W2n, b2n, We_bf, Wn_bf)
        ean_t_gs.append(ean_t_g)
        feats_gs.append(feats_g)

    ea_new = jnp.concatenate(ean_t_gs, axis=1).T
    zeros = jnp.zeros((N_NODES, D_NODE), jnp.float32)
    parts_a = _get_sc_scatter(2, True)(*feats_gs[0:2], srcm[0:2],
                                       node_features, zeros)
    parts_b = _get_sc_scatter(2, False)(*feats_gs[2:4], srcm[2:4],
                                        node_features, zeros)
    parts_c = _get_sc_scatter(1, False)(*feats_gs[4:5], srcm[4:5],
                                        node_features, zeros)
    node_new = _run_combine(parts_a, parts_b, parts_c)
    return (node_new, ea_new)
